# baseline (speedup 1.0000x reference)
import functools

import jax
import jax.numpy as jnp
from jax import lax
from jax.experimental import pallas as pl
from jax.experimental.pallas import tpu as pltpu
from jax.experimental.pallas import tpu_sc as plsc

B = 1024
MAX_LEN = 200
ENTIRE = 196
REGION = 5
EMB = 16
NUM_CLASSES = 1000

NC = 2
NS = 16
NW = NC * NS
B_PER_W = B // NW

VOCAB = 100000
DET_CH = 160
DET_NCHUNK = VOCAB // DET_CH
DET_KK = -(-DET_NCHUNK // NW)


MERGED = REGION * EMB + EMB


@functools.partial(
    pl.kernel,
    out_type=jax.ShapeDtypeStruct((VOCAB * MERGED,), jnp.float32),
    mesh=plsc.VectorSubcoreMesh(core_axis_name="c", subcore_axis_name="s"),
    scratch_types=[
        pltpu.VMEM((DET_CH, REGION * EMB), jnp.float32),
        pltpu.VMEM((DET_CH, EMB), jnp.float32),
        pltpu.VMEM((DET_CH * MERGED,), jnp.float32),
        pltpu.SemaphoreType.DMA,
    ],
)
def _sc_detile(tab_hbm, wtab_hbm, out_hbm, tbuf, wtbuf, fbuf, sem):
    wid = lax.axis_index("s") * NC + lax.axis_index("c")
    d = REGION * EMB

    def do_chunk(kk, carry):
        c = wid + NW * kk

        @pl.when(c < DET_NCHUNK)
        def _():
            cp = pltpu.make_async_copy(tab_hbm.at[pl.ds(c * DET_CH, DET_CH)],
                                       tbuf, sem)
            wcp = pltpu.make_async_copy(wtab_hbm.at[pl.ds(c * DET_CH, DET_CH)],
                                        wtbuf, sem)
            cp.start()
            wcp.start()
            cp.wait()
            wcp.wait()

            def row(r, cr):
                for k in range(d // 16):
                    fbuf[pl.ds(r * MERGED + 16 * k, 16)] = tbuf[r, pl.ds(16 * k, 16)]
                fbuf[pl.ds(r * MERGED + d, 16)] = wtbuf[r]
                return cr

            lax.fori_loop(0, DET_CH, row, 0)
            out = pltpu.make_async_copy(
                fbuf, out_hbm.at[pl.ds(c * DET_CH * MERGED, DET_CH * MERGED)],
                sem)
            out.start()
            out.wait()

        return carry

    lax.fori_loop(0, DET_KK, do_chunk, 0)


def _sc_body(train_hbm, merged_hbm, out_hbm,
             tokens_v, rbuf_a, rbuf_b, nout, sem_a, sem_b):
    wid = lax.axis_index("s") * NC + lax.axis_index("c")
    base = wid * B_PER_W
    pltpu.sync_copy(train_hbm.at[pl.ds(base, B_PER_W)], tokens_v)

    def copies(row, rbuf, sem):
        i0 = tokens_v.at[row, pl.ds(0, 104)]
        i1 = tokens_v.at[row, pl.ds(104, 96)]
        return (
            pltpu.make_async_copy(merged_hbm.at[i0], rbuf.at[pl.ds(0, 104)], sem),
            pltpu.make_async_copy(merged_hbm.at[i1], rbuf.at[pl.ds(104, 96)], sem),
        )

    def fire(row, rbuf, sem):
        for c in copies(row, rbuf, sem):
            c.start()

    def drain(row, rbuf, sem):
        for c in copies(row, rbuf, sem):
            c.wait()

    def compute(bi, rbuf):
        def wvec(w):
            return rbuf[w, pl.ds(REGION * EMB, 16)]

        def group(row, wv, ps, m=None):
            for p in ps:
                pr = rbuf[row, pl.ds(16 * p, 16)] * wv[p]
                m = pr if m is None else jnp.maximum(m, pr)
            return m

        wv = [wvec(p) for p in range(5)]
        acc = group(2, wv, range(5))
        stash = [group(39 * q + 2, wv, range(q, 5)) for q in range(1, 5)]

        def blk_body(blk, acc):
            wv = [wvec(5 * blk + p) for p in range(5)]
            for q in range(5):
                acc = acc + group(blk + 39 * q + 2, wv, range(5))
            return acc

        acc = lax.fori_loop(1, 39, blk_body, acc)

        wv = [wvec(195 + p) for p in range(5)]
        for q in range(4):
            acc = acc + group(39 * (q + 1) + 2, wv, range(q + 1), stash[q])
        acc = acc + group(197, wv, range(5))
        nout[bi] = acc

    last = B_PER_W - 1
    fire(0, rbuf_a, sem_a)

    def do_pair(k, carry):
        g = 2 * k
        fire(jnp.minimum(g + 1, last), rbuf_b, sem_b)
        drain(g, rbuf_a, sem_a)
        compute(g, rbuf_a)
        fire(jnp.minimum(g + 2, last), rbuf_a, sem_a)
        drain(g + 1, rbuf_b, sem_b)
        compute(g + 1, rbuf_b)
        return carry

    lax.fori_loop(0, B_PER_W // 2, do_pair, 0)
    drain(last, rbuf_a, sem_a)
    pltpu.sync_copy(nout, out_hbm.at[pl.ds(base, B_PER_W)])


@functools.partial(
    pl.kernel,
    out_type=jax.ShapeDtypeStruct((B, EMB), jnp.float32),
    mesh=plsc.VectorSubcoreMesh(core_axis_name="c", subcore_axis_name="s"),
    compiler_params=pltpu.CompilerParams(use_tc_tiling_on_sc=False),
    scratch_types=[
        pltpu.VMEM((B_PER_W, MAX_LEN), jnp.int32),
        pltpu.VMEM((MAX_LEN, MERGED), jnp.float32),
        pltpu.VMEM((MAX_LEN, MERGED), jnp.float32),
        pltpu.VMEM((B_PER_W, EMB), jnp.float32),
        pltpu.SemaphoreType.DMA,
        pltpu.SemaphoreType.DMA,
    ],
)
def _sc_nointer(train_hbm, merged_hbm, out_hbm,
                tokens_v, rbuf_a, rbuf_b, nout, sem_a, sem_b):
    _sc_body(train_hbm, merged_hbm, out_hbm,
             tokens_v, rbuf_a, rbuf_b, nout, sem_a, sem_b)


def _mlp_body(x_ref, w1_ref, b1_ref, w2_ref, b2_ref,
              logits_ref, prob_ref, cls_ref):
    x = x_ref[...]
    h = lax.dot_general(x, w1_ref[...], (((1,), (1,)), ((), ())),
                        preferred_element_type=jnp.float32) + b1_ref[...]
    h = jnp.maximum(h, 0.0)
    logits = lax.dot_general(h, w2_ref[...], (((1,), (1,)), ((), ())),
                             preferred_element_type=jnp.float32) + b2_ref[...]
    logits_ref[...] = logits
    mx = jnp.max(logits, axis=1, keepdims=True)
    e = jnp.exp(logits - mx)
    s = jnp.sum(e, axis=1, keepdims=True)
    prob_ref[...] = e / s
    idx = lax.broadcasted_iota(jnp.int32, logits.shape, 1)
    cand = jnp.where(logits == mx, idx, NUM_CLASSES)
    cls_ref[...] = jnp.min(cand, axis=1, keepdims=True)


def kernel(train_input, region_table, word_table, W1, b1, W2, b2):
    merged = _sc_detile(region_table, word_table).reshape(VOCAB, MERGED)
    nointer = _sc_nointer(train_input, merged)

    logits, prob, cls = pl.pallas_call(
        _mlp_body,
        out_shape=[
            jax.ShapeDtypeStruct((B, NUM_CLASSES), jnp.float32),
            jax.ShapeDtypeStruct((B, NUM_CLASSES), jnp.float32),
            jax.ShapeDtypeStruct((B, 1), jnp.int32),
        ],
    )(nointer, W1, b1.reshape(1, -1), W2, b2.reshape(1, -1))
    return (logits, prob, cls.reshape(-1))


# speedup vs baseline: 1.0018x; 1.0018x over previous
import functools

import jax
import jax.numpy as jnp
from jax import lax
from jax.experimental import pallas as pl
from jax.experimental.pallas import tpu as pltpu
from jax.experimental.pallas import tpu_sc as plsc

B = 1024
MAX_LEN = 200
ENTIRE = 196
REGION = 5
EMB = 16
NUM_CLASSES = 1000

NC = 2
NS = 16
NW = NC * NS
B_PER_W = B // NW

VOCAB = 100000
DET_CH = 160
DET_NCHUNK = VOCAB // DET_CH
DET_KK = -(-DET_NCHUNK // NW)


MERGED = REGION * EMB + EMB


@functools.partial(
    pl.kernel,
    out_type=jax.ShapeDtypeStruct((VOCAB * MERGED,), jnp.float32),
    mesh=plsc.VectorSubcoreMesh(core_axis_name="c", subcore_axis_name="s"),
    scratch_types=[
        pltpu.VMEM((DET_CH, REGION * EMB), jnp.float32),
        pltpu.VMEM((DET_CH, EMB), jnp.float32),
        pltpu.VMEM((DET_CH * MERGED,), jnp.float32),
        pltpu.SemaphoreType.DMA,
    ],
)
def _sc_detile(tab_hbm, wtab_hbm, out_hbm, tbuf, wtbuf, fbuf, sem):
    wid = lax.axis_index("s") * NC + lax.axis_index("c")
    d = REGION * EMB

    def do_chunk(kk, carry):
        c = wid + NW * kk

        @pl.when(c < DET_NCHUNK)
        def _():
            cp = pltpu.make_async_copy(tab_hbm.at[pl.ds(c * DET_CH, DET_CH)],
                                       tbuf, sem)
            wcp = pltpu.make_async_copy(wtab_hbm.at[pl.ds(c * DET_CH, DET_CH)],
                                        wtbuf, sem)
            cp.start()
            wcp.start()
            cp.wait()
            wcp.wait()

            def row(r, cr):
                for k in range(d // 16):
                    fbuf[pl.ds(r * MERGED + 16 * k, 16)] = tbuf[r, pl.ds(16 * k, 16)]
                fbuf[pl.ds(r * MERGED + d, 16)] = wtbuf[r]
                return cr

            lax.fori_loop(0, DET_CH, row, 0)
            out = pltpu.make_async_copy(
                fbuf, out_hbm.at[pl.ds(c * DET_CH * MERGED, DET_CH * MERGED)],
                sem)
            out.start()
            out.wait()

        return carry

    lax.fori_loop(0, DET_KK, do_chunk, 0)


def _sc_body(train_hbm, merged_hbm, out_hbm,
             tokens_v, rbuf_a, rbuf_b, nout, sem_a, sem_b):
    wid = lax.axis_index("s") * NC + lax.axis_index("c")
    base = wid * B_PER_W
    pltpu.sync_copy(train_hbm.at[pl.ds(base, B_PER_W)], tokens_v)

    def copies(row, rbuf, sem):
        out = []
        for (o, n) in ((0, 56), (56, 48), (104, 48), (152, 48)):
            idx = tokens_v.at[row, pl.ds(o, n)]
            out.append(pltpu.make_async_copy(merged_hbm.at[idx],
                                             rbuf.at[pl.ds(o, n)], sem))
        return out

    def fire(row, rbuf, sem):
        for c in copies(row, rbuf, sem):
            c.start()

    def drain(row, rbuf, sem):
        for c in copies(row, rbuf, sem):
            c.wait()

    def compute(bi, rbuf):
        def wvec(w):
            return rbuf[w, pl.ds(REGION * EMB, 16)]

        def group(row, wv, ps, m=None):
            for p in ps:
                pr = rbuf[row, pl.ds(16 * p, 16)] * wv[p]
                m = pr if m is None else jnp.maximum(m, pr)
            return m

        wv = [wvec(p) for p in range(5)]
        acc = group(2, wv, range(5))
        stash = [group(39 * q + 2, wv, range(q, 5)) for q in range(1, 5)]

        def blk_body(blk, acc):
            wv = [wvec(5 * blk + p) for p in range(5)]
            for q in range(5):
                acc = acc + group(blk + 39 * q + 2, wv, range(5))
            return acc

        acc = lax.fori_loop(1, 39, blk_body, acc)

        wv = [wvec(195 + p) for p in range(5)]
        for q in range(4):
            acc = acc + group(39 * (q + 1) + 2, wv, range(q + 1), stash[q])
        acc = acc + group(197, wv, range(5))
        nout[bi] = acc

    last = B_PER_W - 1
    fire(0, rbuf_a, sem_a)

    def do_pair(k, carry):
        g = 2 * k
        fire(jnp.minimum(g + 1, last), rbuf_b, sem_b)
        drain(g, rbuf_a, sem_a)
        compute(g, rbuf_a)
        fire(jnp.minimum(g + 2, last), rbuf_a, sem_a)
        drain(g + 1, rbuf_b, sem_b)
        compute(g + 1, rbuf_b)
        return carry

    lax.fori_loop(0, B_PER_W // 2, do_pair, 0)
    drain(last, rbuf_a, sem_a)
    pltpu.sync_copy(nout, out_hbm.at[pl.ds(base, B_PER_W)])


@functools.partial(
    pl.kernel,
    out_type=jax.ShapeDtypeStruct((B, EMB), jnp.float32),
    mesh=plsc.VectorSubcoreMesh(core_axis_name="c", subcore_axis_name="s"),
    compiler_params=pltpu.CompilerParams(use_tc_tiling_on_sc=False),
    scratch_types=[
        pltpu.VMEM((B_PER_W, MAX_LEN), jnp.int32),
        pltpu.VMEM((MAX_LEN, MERGED), jnp.float32),
        pltpu.VMEM((MAX_LEN, MERGED), jnp.float32),
        pltpu.VMEM((B_PER_W, EMB), jnp.float32),
        pltpu.SemaphoreType.DMA,
        pltpu.SemaphoreType.DMA,
    ],
)
def _sc_nointer(train_hbm, merged_hbm, out_hbm,
                tokens_v, rbuf_a, rbuf_b, nout, sem_a, sem_b):
    _sc_body(train_hbm, merged_hbm, out_hbm,
             tokens_v, rbuf_a, rbuf_b, nout, sem_a, sem_b)


def _mlp_body(x_ref, w1_ref, b1_ref, w2_ref, b2_ref,
              logits_ref, prob_ref, cls_ref):
    x = x_ref[...]
    h = lax.dot_general(x, w1_ref[...], (((1,), (1,)), ((), ())),
                        preferred_element_type=jnp.float32) + b1_ref[...]
    h = jnp.maximum(h, 0.0)
    logits = lax.dot_general(h, w2_ref[...], (((1,), (1,)), ((), ())),
                             preferred_element_type=jnp.float32) + b2_ref[...]
    logits_ref[...] = logits
    mx = jnp.max(logits, axis=1, keepdims=True)
    e = jnp.exp(logits - mx)
    s = jnp.sum(e, axis=1, keepdims=True)
    prob_ref[...] = e / s
    idx = lax.broadcasted_iota(jnp.int32, logits.shape, 1)
    cand = jnp.where(logits == mx, idx, NUM_CLASSES)
    cls_ref[...] = jnp.min(cand, axis=1, keepdims=True)


def kernel(train_input, region_table, word_table, W1, b1, W2, b2):
    merged = _sc_detile(region_table, word_table).reshape(VOCAB, MERGED)
    nointer = _sc_nointer(train_input, merged)

    logits, prob, cls = pl.pallas_call(
        _mlp_body,
        out_shape=[
            jax.ShapeDtypeStruct((B, NUM_CLASSES), jnp.float32),
            jax.ShapeDtypeStruct((B, NUM_CLASSES), jnp.float32),
            jax.ShapeDtypeStruct((B, 1), jnp.int32),
        ],
    )(nointer, W1, b1.reshape(1, -1), W2, b2.reshape(1, -1))
    return (logits, prob, cls.reshape(-1))


# speedup vs baseline: 1.2034x; 1.2012x over previous
import functools

import jax
import jax.numpy as jnp
from jax import lax
from jax.experimental import pallas as pl
from jax.experimental.pallas import tpu as pltpu
from jax.experimental.pallas import tpu_sc as plsc

B = 1024
MAX_LEN = 200
ENTIRE = 196
REGION = 5
EMB = 16
NUM_CLASSES = 1000

NC = 2
NS = 16
NW = NC * NS
B_PER_W = B // NW

VOCAB = 100000
DET_CH = 160
DET_NCHUNK = VOCAB // DET_CH
DET_KK = -(-DET_NCHUNK // NW)


@functools.partial(
    pl.kernel,
    out_type=jax.ShapeDtypeStruct((VOCAB * REGION * EMB,), jnp.float32),
    mesh=plsc.VectorSubcoreMesh(core_axis_name="c", subcore_axis_name="s"),
    scratch_types=[
        pltpu.VMEM((DET_CH, REGION * EMB), jnp.float32),
        pltpu.VMEM((DET_CH * REGION * EMB,), jnp.float32),
        pltpu.SemaphoreType.DMA,
    ],
)
def _sc_detile(tab_hbm, out_hbm, tbuf, fbuf, sem):
    wid = lax.axis_index("s") * NC + lax.axis_index("c")
    d = REGION * EMB

    def do_chunk(kk, carry):
        c = wid + NW * kk

        @pl.when(c < DET_NCHUNK)
        def _():
            cp = pltpu.make_async_copy(tab_hbm.at[pl.ds(c * DET_CH, DET_CH)],
                                       tbuf, sem)
            cp.start()
            cp.wait()

            def row(r, cr):
                for k in range(d // 16):
                    fbuf[pl.ds(r * d + 16 * k, 16)] = tbuf[r, pl.ds(16 * k, 16)]
                return cr

            lax.fori_loop(0, DET_CH, row, 0)
            out = pltpu.make_async_copy(
                fbuf, out_hbm.at[pl.ds(c * DET_CH * d, DET_CH * d)], sem)
            out.start()
            out.wait()

        return carry

    lax.fori_loop(0, DET_KK, do_chunk, 0)


def _sc_body(train_hbm, region_hbm, word_hbm, out_hbm,
             tokens_v, rbuf_a, wbuf_a, rbuf_b, wbuf_b, nout, sem_a, sem_b):
    wid = lax.axis_index("s") * NC + lax.axis_index("c")
    base = wid * B_PER_W
    pltpu.sync_copy(train_hbm.at[pl.ds(base, B_PER_W)], tokens_v)

    def copies(row, rbuf, wbuf, sem):
        i0 = tokens_v.at[row, pl.ds(0, 104)]
        i1 = tokens_v.at[row, pl.ds(104, 96)]
        return (
            pltpu.make_async_copy(region_hbm.at[i0], rbuf.at[pl.ds(0, 104)], sem),
            pltpu.make_async_copy(region_hbm.at[i1], rbuf.at[pl.ds(104, 96)], sem),
            pltpu.make_async_copy(word_hbm.at[i0], wbuf.at[pl.ds(0, 104)], sem),
            pltpu.make_async_copy(word_hbm.at[i1], wbuf.at[pl.ds(104, 96)], sem),
        )

    def fire(row, rbuf, wbuf, sem):
        for c in copies(row, rbuf, wbuf, sem):
            c.start()

    def drain(row, rbuf, wbuf, sem):
        for c in copies(row, rbuf, wbuf, sem):
            c.wait()

    def compute(bi, rbuf, wbuf):
        def wvec(w):
            return wbuf[w]

        def group(row, wv, ps, m=None):
            for p in ps:
                pr = rbuf[row, pl.ds(16 * p, 16)] * wv[p]
                m = pr if m is None else jnp.maximum(m, pr)
            return m

        wv = [wvec(p) for p in range(5)]
        acc = group(2, wv, range(5))
        stash = [group(39 * q + 2, wv, range(q, 5)) for q in range(1, 5)]

        def blk_body(blk, acc):
            wv = [wvec(5 * blk + p) for p in range(5)]
            for q in range(5):
                acc = acc + group(blk + 39 * q + 2, wv, range(5))
            return acc

        acc = lax.fori_loop(1, 39, blk_body, acc)

        wv = [wvec(195 + p) for p in range(5)]
        for q in range(4):
            acc = acc + group(39 * (q + 1) + 2, wv, range(q + 1), stash[q])
        acc = acc + group(197, wv, range(5))
        nout[bi] = acc

    last = B_PER_W - 1
    fire(0, rbuf_a, wbuf_a, sem_a)

    def do_pair(k, carry):
        g = 2 * k
        fire(jnp.minimum(g + 1, last), rbuf_b, wbuf_b, sem_b)
        drain(g, rbuf_a, wbuf_a, sem_a)
        compute(g, rbuf_a, wbuf_a)
        fire(jnp.minimum(g + 2, last), rbuf_a, wbuf_a, sem_a)
        drain(g + 1, rbuf_b, wbuf_b, sem_b)
        compute(g + 1, rbuf_b, wbuf_b)
        return carry

    lax.fori_loop(0, B_PER_W // 2, do_pair, 0)
    drain(last, rbuf_a, wbuf_a, sem_a)
    pltpu.sync_copy(nout, out_hbm.at[pl.ds(base, B_PER_W)])


@functools.partial(
    pl.kernel,
    out_type=jax.ShapeDtypeStruct((B, EMB), jnp.float32),
    mesh=plsc.VectorSubcoreMesh(core_axis_name="c", subcore_axis_name="s"),
    compiler_params=pltpu.CompilerParams(use_tc_tiling_on_sc=False),
    scratch_types=[
        pltpu.VMEM((B_PER_W, MAX_LEN), jnp.int32),
        pltpu.VMEM((MAX_LEN, REGION * EMB), jnp.float32),
        pltpu.VMEM((MAX_LEN, EMB), jnp.float32),
        pltpu.VMEM((MAX_LEN, REGION * EMB), jnp.float32),
        pltpu.VMEM((MAX_LEN, EMB), jnp.float32),
        pltpu.VMEM((B_PER_W, EMB), jnp.float32),
        pltpu.SemaphoreType.DMA,
        pltpu.SemaphoreType.DMA,
    ],
)
def _sc_nointer(train_hbm, region_hbm, word_hbm, out_hbm,
                tokens_v, rbuf_a, wbuf_a, rbuf_b, wbuf_b, nout, sem_a, sem_b):
    _sc_body(train_hbm, region_hbm, word_hbm, out_hbm,
             tokens_v, rbuf_a, wbuf_a, rbuf_b, wbuf_b, nout, sem_a, sem_b)


def _mlp_body(x_ref, w1_ref, b1_ref, w2_ref, b2_ref,
              logits_ref, prob_ref, cls_ref):
    x = x_ref[...]
    h = lax.dot_general(x, w1_ref[...], (((1,), (1,)), ((), ())),
                        preferred_element_type=jnp.float32) + b1_ref[...]
    h = jnp.maximum(h, 0.0)
    logits = lax.dot_general(h, w2_ref[...], (((1,), (1,)), ((), ())),
                             preferred_element_type=jnp.float32) + b2_ref[...]
    logits_ref[...] = logits
    mx = jnp.max(logits, axis=1, keepdims=True)
    e = jnp.exp(logits - mx)
    s = jnp.sum(e, axis=1, keepdims=True)
    prob_ref[...] = e / s
    idx = lax.broadcasted_iota(jnp.int32, logits.shape, 1)
    cand = jnp.where(logits == mx, idx, NUM_CLASSES)
    cls_ref[...] = jnp.min(cand, axis=1, keepdims=True)


def kernel(train_input, region_table, word_table, W1, b1, W2, b2):
    region_lin = _sc_detile(region_table).reshape(VOCAB, REGION * EMB)
    nointer = _sc_nointer(train_input, region_lin, word_table)

    logits, prob, cls = pl.pallas_call(
        _mlp_body,
        out_shape=[
            jax.ShapeDtypeStruct((B, NUM_CLASSES), jnp.float32),
            jax.ShapeDtypeStruct((B, NUM_CLASSES), jnp.float32),
            jax.ShapeDtypeStruct((B, 1), jnp.int32),
        ],
    )(nointer, W1, b1.reshape(1, -1), W2, b2.reshape(1, -1))
    return (logits, prob, cls.reshape(-1))
